# baseline (speedup 1.0000x reference)
import functools

import jax
import jax.numpy as jnp
from jax import lax
from jax.experimental import pallas as pl
from jax.experimental.pallas import tpu as pltpu
from jax.experimental.pallas import tpu_sc as plsc

K = 16
L = 16
NC = 2
NS = 16
NW = NC * NS
G = 80
C = 128
NPAD = 10240
CPAD = 128


def _linear_kernel(x_ref, w_ref, h_ref):
    h_ref[...] = lax.dot_general(
        x_ref[...], w_ref[...],
        (((1,), (1,)), ((), ())),
        preferred_element_type=jnp.float32,
    )


def _chunkmax_kernel(x_ref, m_ref):
    x = x_ref[...]
    br, n = x.shape
    pad = jnp.full((br, NPAD - n), -jnp.inf, jnp.float32)
    y = jnp.concatenate([x, pad], axis=1).reshape(br, G, C)
    m_ref[...] = jnp.max(y, axis=1)


def _merge_top16(tv, ti, sk, si):
    rb = lax.rev(sk, (0,))
    rbi = lax.rev(si, (0,))
    keep = tv >= rb
    mv = jnp.maximum(tv, rb)
    mi = jnp.where(keep, ti, rbi)
    return plsc.sort_key_val(mv, mi)


def _iota16():
    return lax.iota(jnp.int32, L)


def _sc_body(logits_hbm, m_hbm, h_hbm, att_hbm, hp_hbm,
             row_v, m_v, out_v, hgat_v, hp_v, prev_v, soft_v,
             rsem, msem, osem, hpsem, hsem):
    n = 10000
    wid = lax.axis_index("s") * NC + lax.axis_index("c")
    n_lo = n // NW
    n_extra = n - n_lo * NW
    cnt = jnp.where(wid < n_extra, n_lo + 1, n_lo)
    base = jnp.where(wid < n_extra, wid * (n_lo + 1),
                     n_extra * (n_lo + 1) + (wid - n_extra) * n_lo)
    last = n - 1

    def zero_body(i, _):
        out_v[0, pl.ds(i * L, L)] = jnp.zeros((L,), jnp.float32)
        out_v[1, pl.ds(i * L, L)] = jnp.zeros((L,), jnp.float32)
        return 0
    lax.fori_loop(0, n // L, zero_body, 0)

    prev_v[0, :] = jnp.zeros((L,), jnp.int32)
    prev_v[1, :] = jnp.zeros((L,), jnp.int32)

    zeros16f = jnp.zeros((L,), jnp.float32)
    iota = _iota16()

    pltpu.async_copy(logits_hbm.at[base], row_v.at[0], rsem)
    pltpu.async_copy(m_hbm.at[base], m_v.at[0], msem)

    def row_body(t, _):
        r = base + t
        slot = lax.rem(t, 2)
        nslot = 1 - slot
        rnext = jnp.minimum(base + t + 1, last)

        pltpu.make_async_copy(logits_hbm.at[r], row_v.at[slot], rsem).wait()
        pltpu.make_async_copy(m_hbm.at[r], m_v.at[slot], msem).wait()

        @pl.when(t + 1 < cnt)
        def _():
            pltpu.async_copy(logits_hbm.at[rnext], row_v.at[nslot], rsem)
            pltpu.async_copy(m_hbm.at[rnext], m_v.at[nslot], msem)

        slotv = jnp.full((L,), slot, jnp.int32)

        tv = jnp.full((L,), -jnp.inf, jnp.float32)
        ti = jnp.zeros((L,), jnp.int32)
        for v in range(CPAD // L):
            k = m_v[slot, pl.ds(v * L, L)]
            sk, si = plsc.sort_key_val(k, iota + v * L)
            tv, ti = _merge_top16(tv, ti, sk, si)

        top_v = jnp.full((L,), -jnp.inf, jnp.float32)
        top_i = jnp.zeros((L,), jnp.int32)

        def chunk_body(s, carry):
            cv, ci = carry
            cid = jnp.take_along_axis(ti, jnp.full((L,), s, jnp.int32), axis=0)
            for v in range(G // L):
                idx = cid + (C * (v * L)) + C * iota
                if (v + 1) * L * C >= n:
                    safe = jnp.minimum(idx, n - 1)
                    val = plsc.load_gather(row_v, [slotv, safe])
                    val = jnp.where(idx < n, val, -jnp.inf)
                else:
                    val = plsc.load_gather(row_v, [slotv, idx])
                sk, si = plsc.sort_key_val(val, idx)
                cv, ci = _merge_top16(cv, ci, sk, si)
            return cv, ci

        top_v, top_i = lax.fori_loop(0, L, chunk_body, (top_v, top_i))

        mx = jnp.max(top_v)
        e = jnp.exp(top_v - mx)
        ssum = jnp.sum(e)
        soft = e / ssum

        pltpu.async_copy(h_hbm.at[top_i], hgat_v.at[slot], hsem)
        soft_v[slot, :] = soft

        @pl.when(t >= 2)
        def _():
            pltpu.make_async_copy(out_v.at[slot], att_hbm.at[r], osem).wait()
        prev_i = prev_v[slot, :]
        plsc.store_scatter(out_v, [slotv, prev_i], zeros16f)
        plsc.store_scatter(out_v, [slotv, top_i], soft)
        prev_v[slot, :] = top_i
        pltpu.async_copy(out_v.at[slot], att_hbm.at[r], osem)

        @pl.when(t >= 1)
        def _():
            pltpu.make_async_copy(h_hbm.at[top_i], hgat_v.at[nslot],
                                  hsem).wait()
            softp = soft_v[nslot, :]
            @pl.when(t >= 3)
            def _():
                pltpu.make_async_copy(hp_v.at[nslot], hp_hbm.at[r],
                                      hpsem).wait()
            accs = [jnp.zeros((L,), jnp.float32) for _ in range(8)]
            for k2 in range(K):
                w = jnp.take_along_axis(softp, jnp.full((L,), k2, jnp.int32),
                                        axis=0)
                for j in range(8):
                    accs[j] = accs[j] + w * hgat_v[nslot, k2, pl.ds(j * L, L)]
            for j in range(8):
                hp_v[nslot, pl.ds(j * L, L)] = accs[j]
            pltpu.async_copy(hp_v.at[nslot], hp_hbm.at[r - 1], hpsem)
        return 0

    lax.fori_loop(0, cnt, row_body, 0)

    lslot = lax.rem(cnt - 1, 2)
    rlast = base + cnt - 1
    pltpu.make_async_copy(h_hbm.at[jnp.zeros((L,), jnp.int32)],
                          hgat_v.at[lslot], hsem).wait()
    pltpu.make_async_copy(hp_v.at[lslot], hp_hbm.at[rlast], hpsem).wait()
    softp = soft_v[lslot, :]
    accs = [jnp.zeros((L,), jnp.float32) for _ in range(8)]
    for k2 in range(K):
        w = jnp.take_along_axis(softp, jnp.full((L,), k2, jnp.int32), axis=0)
        for j in range(8):
            accs[j] = accs[j] + w * hgat_v[lslot, k2, pl.ds(j * L, L)]
    for j in range(8):
        hp_v[lslot, pl.ds(j * L, L)] = accs[j]
    pltpu.async_copy(hp_v.at[lslot], hp_hbm.at[rlast], hpsem)

    pltpu.make_async_copy(out_v.at[0], att_hbm.at[base], osem).wait()
    pltpu.make_async_copy(out_v.at[0], att_hbm.at[base], osem).wait()
    pltpu.make_async_copy(hp_v.at[0], hp_hbm.at[base], hpsem).wait()
    pltpu.make_async_copy(hp_v.at[0], hp_hbm.at[base], hpsem).wait()


@functools.partial(jax.jit, static_argnames=())
def kernel(input, attention_logits, W):
    n_in, d_in = input.shape
    rows, n = attention_logits.shape
    d_out = W.shape[0]

    h = pl.pallas_call(
        _linear_kernel,
        out_shape=jax.ShapeDtypeStruct((n_in, d_out), jnp.float32),
    )(input, W)

    br = 400
    m = pl.pallas_call(
        _chunkmax_kernel,
        grid=(rows // br,),
        in_specs=[pl.BlockSpec((br, n), lambda i: (i, 0))],
        out_specs=pl.BlockSpec((br, CPAD), lambda i: (i, 0)),
        out_shape=jax.ShapeDtypeStruct((rows, CPAD), jnp.float32),
    )(attention_logits)

    mesh = plsc.VectorSubcoreMesh(core_axis_name="c", subcore_axis_name="s")
    att, hp = pl.kernel(
        _sc_body,
        out_type=[
            jax.ShapeDtypeStruct((rows, n), jnp.float32),
            jax.ShapeDtypeStruct((rows, d_out), jnp.float32),
        ],
        mesh=mesh,
        scratch_types=[
            pltpu.VMEM((2, n), jnp.float32),
            pltpu.VMEM((2, CPAD), jnp.float32),
            pltpu.VMEM((2, n), jnp.float32),
            pltpu.VMEM((2, K, d_out), jnp.float32),
            pltpu.VMEM((2, d_out), jnp.float32),
            pltpu.VMEM((2, L), jnp.int32),
            pltpu.VMEM((2, L), jnp.float32),
            pltpu.SemaphoreType.DMA,
            pltpu.SemaphoreType.DMA,
            pltpu.SemaphoreType.DMA,
            pltpu.SemaphoreType.DMA,
            pltpu.SemaphoreType.DMA,
        ],
        compiler_params=pltpu.CompilerParams(needs_layout_passes=False),
    )(attention_logits, m, h)

    oc = rows // n
    out = hp.reshape(oc, n, d_out).transpose(1, 0, 2).reshape(n, oc * d_out)
    return out, att


# speedup vs baseline: 1.2859x; 1.2859x over previous
import functools

import jax
import jax.numpy as jnp
from jax import lax
from jax.experimental import pallas as pl
from jax.experimental.pallas import tpu as pltpu
from jax.experimental.pallas import tpu_sc as plsc

K = 16
L = 16
NC = 2
NS = 16
NW = NC * NS
C = 128
DEPTH = 78
CPAD = 128


def _linear_kernel(x_ref, w_ref, h_ref):
    h_ref[...] = lax.dot_general(
        x_ref[...], w_ref[...],
        (((1,), (1,)), ((), ())),
        preferred_element_type=jnp.float32,
    )


def _chunkmax_kernel(x_ref, m_ref):
    x = x_ref[...]
    br, n = x.shape
    m = x[:, 0:C]
    for v in range(1, n // C):
        m = jnp.maximum(m, x[:, C * v:C * (v + 1)])
    tail = jnp.concatenate(
        [x[:, (n // C) * C:],
         jnp.full((br, C - n % C), -jnp.inf, jnp.float32)], axis=1)
    m_ref[...] = jnp.maximum(m, tail)


def _merge_top16(tv, ti, sk, si):
    rb = lax.rev(sk, (0,))
    rbi = lax.rev(si, (0,))
    keep = tv >= rb
    mv = jnp.maximum(tv, rb)
    mi = jnp.where(keep, ti, rbi)
    return plsc.sort_key_val(mv, mi)


def _iota16():
    return lax.iota(jnp.int32, L)


def _sc_body(logits_hbm, m_hbm, h_hbm, att_hbm, hp_hbm,
             row_v, m_v, out_v, hgat_v, hp_v, prev_v, soft_v,
             rsem, msem, osem, hpsem, hsem):
    n = 10000
    wid = lax.axis_index("s") * NC + lax.axis_index("c")
    n_lo = n // NW
    n_extra = n - n_lo * NW
    cnt = jnp.where(wid < n_extra, n_lo + 1, n_lo)
    base = jnp.where(wid < n_extra, wid * (n_lo + 1),
                     n_extra * (n_lo + 1) + (wid - n_extra) * n_lo)
    last = n - 1

    def zero_body(i, _):
        out_v[0, pl.ds(i * L, L)] = jnp.zeros((L,), jnp.float32)
        out_v[1, pl.ds(i * L, L)] = jnp.zeros((L,), jnp.float32)
        return 0
    lax.fori_loop(0, n // L, zero_body, 0)

    prev_v[0, :] = jnp.zeros((L,), jnp.int32)
    prev_v[1, :] = jnp.zeros((L,), jnp.int32)

    zeros16f = jnp.zeros((L,), jnp.float32)
    iota = _iota16()

    pltpu.async_copy(logits_hbm.at[base], row_v.at[0], rsem)
    pltpu.async_copy(m_hbm.at[base], m_v.at[0], msem)

    def row_body(t, _):
        r = base + t
        slot = lax.rem(t, 2)
        nslot = 1 - slot
        rnext = jnp.minimum(base + t + 1, last)

        pltpu.make_async_copy(logits_hbm.at[r], row_v.at[slot], rsem).wait()
        pltpu.make_async_copy(m_hbm.at[r], m_v.at[slot], msem).wait()

        @pl.when(t + 1 < cnt)
        def _():
            pltpu.async_copy(logits_hbm.at[rnext], row_v.at[nslot], rsem)
            pltpu.async_copy(m_hbm.at[rnext], m_v.at[nslot], msem)

        slotv = jnp.full((L,), slot, jnp.int32)

        tv = jnp.full((L,), -jnp.inf, jnp.float32)
        ti = jnp.zeros((L,), jnp.int32)
        for v in range(CPAD // L):
            k = m_v[slot, pl.ds(v * L, L)]
            sk, si = plsc.sort_key_val(k, iota + v * L)
            tv, ti = _merge_top16(tv, ti, sk, si)

        top_v = jnp.full((L,), -jnp.inf, jnp.float32)
        top_i = jnp.zeros((L,), jnp.int32)

        def chunk_body(j, carry):
            cv, ci = carry
            idx = ti + C * j
            val = plsc.load_gather(row_v, [slotv, idx])
            sk, si = plsc.sort_key_val(val, idx)
            cv, ci = _merge_top16(cv, ci, sk, si)
            return cv, ci

        top_v, top_i = lax.fori_loop(0, DEPTH, chunk_body, (top_v, top_i))
        idx = ti + C * DEPTH
        safe = jnp.minimum(idx, n - 1)
        val = plsc.load_gather(row_v, [slotv, safe])
        val = jnp.where(idx < n, val, -jnp.inf)
        sk, si = plsc.sort_key_val(val, idx)
        top_v, top_i = _merge_top16(top_v, top_i, sk, si)

        mx = jnp.max(top_v)
        e = jnp.exp(top_v - mx)
        ssum = jnp.sum(e)
        soft = e / ssum

        pltpu.async_copy(h_hbm.at[top_i], hgat_v.at[slot], hsem)
        soft_v[slot, :] = soft

        @pl.when(t >= 2)
        def _():
            pltpu.make_async_copy(out_v.at[slot], att_hbm.at[r], osem).wait()
        prev_i = prev_v[slot, :]
        plsc.store_scatter(out_v, [slotv, prev_i], zeros16f)
        plsc.store_scatter(out_v, [slotv, top_i], soft)
        prev_v[slot, :] = top_i
        pltpu.async_copy(out_v.at[slot], att_hbm.at[r], osem)

        @pl.when(t >= 1)
        def _():
            pltpu.make_async_copy(h_hbm.at[top_i], hgat_v.at[nslot],
                                  hsem).wait()
            softp = soft_v[nslot, :]
            @pl.when(t >= 3)
            def _():
                pltpu.make_async_copy(hp_v.at[nslot], hp_hbm.at[r],
                                      hpsem).wait()
            accs = [jnp.zeros((L,), jnp.float32) for _ in range(8)]
            for k2 in range(K):
                w = jnp.take_along_axis(softp, jnp.full((L,), k2, jnp.int32),
                                        axis=0)
                for j in range(8):
                    accs[j] = accs[j] + w * hgat_v[nslot, k2, pl.ds(j * L, L)]
            for j in range(8):
                hp_v[nslot, pl.ds(j * L, L)] = accs[j]
            pltpu.async_copy(hp_v.at[nslot], hp_hbm.at[r - 1], hpsem)
        return 0

    lax.fori_loop(0, cnt, row_body, 0)

    lslot = lax.rem(cnt - 1, 2)
    rlast = base + cnt - 1
    pltpu.make_async_copy(h_hbm.at[jnp.zeros((L,), jnp.int32)],
                          hgat_v.at[lslot], hsem).wait()
    pltpu.make_async_copy(hp_v.at[lslot], hp_hbm.at[rlast], hpsem).wait()
    softp = soft_v[lslot, :]
    accs = [jnp.zeros((L,), jnp.float32) for _ in range(8)]
    for k2 in range(K):
        w = jnp.take_along_axis(softp, jnp.full((L,), k2, jnp.int32), axis=0)
        for j in range(8):
            accs[j] = accs[j] + w * hgat_v[lslot, k2, pl.ds(j * L, L)]
    for j in range(8):
        hp_v[lslot, pl.ds(j * L, L)] = accs[j]
    pltpu.async_copy(hp_v.at[lslot], hp_hbm.at[rlast], hpsem)

    pltpu.make_async_copy(out_v.at[0], att_hbm.at[base], osem).wait()
    pltpu.make_async_copy(out_v.at[0], att_hbm.at[base], osem).wait()
    pltpu.make_async_copy(hp_v.at[0], hp_hbm.at[base], hpsem).wait()
    pltpu.make_async_copy(hp_v.at[0], hp_hbm.at[base], hpsem).wait()


@functools.partial(jax.jit, static_argnames=())
def kernel(input, attention_logits, W):
    n_in, d_in = input.shape
    rows, n = attention_logits.shape
    d_out = W.shape[0]

    h = pl.pallas_call(
        _linear_kernel,
        out_shape=jax.ShapeDtypeStruct((n_in, d_out), jnp.float32),
    )(input, W)

    br = 200
    m = pl.pallas_call(
        _chunkmax_kernel,
        grid=(rows // br,),
        in_specs=[pl.BlockSpec((br, n), lambda i: (i, 0))],
        out_specs=pl.BlockSpec((br, CPAD), lambda i: (i, 0)),
        out_shape=jax.ShapeDtypeStruct((rows, CPAD), jnp.float32),
    )(attention_logits)

    mesh = plsc.VectorSubcoreMesh(core_axis_name="c", subcore_axis_name="s")
    att, hp = pl.kernel(
        _sc_body,
        out_type=[
            jax.ShapeDtypeStruct((rows, n), jnp.float32),
            jax.ShapeDtypeStruct((rows, d_out), jnp.float32),
        ],
        mesh=mesh,
        scratch_types=[
            pltpu.VMEM((2, n), jnp.float32),
            pltpu.VMEM((2, CPAD), jnp.float32),
            pltpu.VMEM((2, n), jnp.float32),
            pltpu.VMEM((2, K, d_out), jnp.float32),
            pltpu.VMEM((2, d_out), jnp.float32),
            pltpu.VMEM((2, L), jnp.int32),
            pltpu.VMEM((2, L), jnp.float32),
            pltpu.SemaphoreType.DMA,
            pltpu.SemaphoreType.DMA,
            pltpu.SemaphoreType.DMA,
            pltpu.SemaphoreType.DMA,
            pltpu.SemaphoreType.DMA,
        ],
        compiler_params=pltpu.CompilerParams(needs_layout_passes=False),
    )(attention_logits, m, h)

    oc = rows // n
    out = hp.reshape(oc, n, d_out).transpose(1, 0, 2).reshape(n, oc * d_out)
    return out, att


# speedup vs baseline: 1.6127x; 1.2542x over previous
import functools

import jax
import jax.numpy as jnp
from jax import lax
from jax.experimental import pallas as pl
from jax.experimental.pallas import tpu as pltpu
from jax.experimental.pallas import tpu_sc as plsc

K = 16
L = 16
NC = 2
NS = 16
NW = NC * NS
C = 128
DEPTH = 78
CPAD = 128


def _linear_kernel(x_ref, w_ref, h_ref):
    h_ref[...] = lax.dot_general(
        x_ref[...], w_ref[...],
        (((1,), (1,)), ((), ())),
        preferred_element_type=jnp.float32,
    )


def _chunkmax_kernel(x_ref, m_ref):
    x = x_ref[...]
    br, n = x.shape
    m = x[:, 0:C]
    for v in range(1, n // C):
        m = jnp.maximum(m, x[:, C * v:C * (v + 1)])
    tail = jnp.concatenate(
        [x[:, (n // C) * C:],
         jnp.full((br, C - n % C), -jnp.inf, jnp.float32)], axis=1)
    m_ref[...] = jnp.maximum(m, tail)


def _merge_top16(tv, ti, sk, si):
    rb = lax.rev(sk, (0,))
    rbi = lax.rev(si, (0,))
    keep = tv >= rb
    mv = jnp.maximum(tv, rb)
    mi = jnp.where(keep, ti, rbi)
    return plsc.sort_key_val(mv, mi)


def _iota16():
    return lax.iota(jnp.int32, L)


def _sc_body(logits_hbm, m_hbm, h_hbm, att_hbm, hp_hbm,
             row_v, m_v, out_v, hgat_v, hp_v, prev_v, soft_v,
             rsem, msem, osem, hpsem, hsem):
    n = 10000
    wid = lax.axis_index("s") * NC + lax.axis_index("c")
    n_lo = n // NW
    n_extra = n - n_lo * NW
    cnt = jnp.where(wid < n_extra, n_lo + 1, n_lo)
    base = jnp.where(wid < n_extra, wid * (n_lo + 1),
                     n_extra * (n_lo + 1) + (wid - n_extra) * n_lo)
    last = n - 1

    def zero_body(i, _):
        out_v[0, pl.ds(i * L, L)] = jnp.zeros((L,), jnp.float32)
        out_v[1, pl.ds(i * L, L)] = jnp.zeros((L,), jnp.float32)
        return 0
    lax.fori_loop(0, n // L, zero_body, 0)

    prev_v[0, :] = jnp.zeros((L,), jnp.int32)
    prev_v[1, :] = jnp.zeros((L,), jnp.int32)

    zeros16f = jnp.zeros((L,), jnp.float32)
    iota = _iota16()

    pltpu.async_copy(logits_hbm.at[base], row_v.at[0], rsem)
    pltpu.async_copy(m_hbm.at[base], m_v.at[0], msem)

    def row_body(t, _):
        r = base + t
        slot = lax.rem(t, 2)
        nslot = 1 - slot
        rnext = jnp.minimum(base + t + 1, last)

        pltpu.make_async_copy(logits_hbm.at[r], row_v.at[slot], rsem).wait()
        pltpu.make_async_copy(m_hbm.at[r], m_v.at[slot], msem).wait()

        @pl.when(t + 1 < cnt)
        def _():
            pltpu.async_copy(logits_hbm.at[rnext], row_v.at[nslot], rsem)
            pltpu.async_copy(m_hbm.at[rnext], m_v.at[nslot], msem)

        slotv = jnp.full((L,), slot, jnp.int32)

        leaves = []
        for v in range(CPAD // L):
            k = m_v[slot, pl.ds(v * L, L)]
            leaves.append(plsc.sort_key_val(k, iota + v * L))
        while len(leaves) > 1:
            nxt = [_merge_top16(*leaves[i], *leaves[i + 1])
                   for i in range(0, len(leaves), 2)]
            leaves = nxt
        tv, ti = leaves[0]

        top_v = jnp.full((L,), -jnp.inf, jnp.float32)
        top_i = jnp.zeros((L,), jnp.int32)

        def leaf(j):
            idx = ti + C * j
            val = plsc.load_gather(row_v, [slotv, idx])
            return plsc.sort_key_val(val, idx)

        def chunk_body(t2, carry):
            cv, ci = carry
            j0 = 4 * t2
            s0 = leaf(j0)
            s1 = leaf(j0 + 1)
            s2 = leaf(j0 + 2)
            s3 = leaf(j0 + 3)
            p01 = _merge_top16(*s0, *s1)
            p23 = _merge_top16(*s2, *s3)
            q = _merge_top16(*p01, *p23)
            cv, ci = _merge_top16(cv, ci, *q)
            return cv, ci

        top_v, top_i = lax.fori_loop(0, DEPTH // 4, chunk_body,
                                     (top_v, top_i))
        s0 = leaf(DEPTH - 2)
        s1 = leaf(DEPTH - 1)
        idx = ti + C * DEPTH
        safe = jnp.minimum(idx, n - 1)
        val = plsc.load_gather(row_v, [slotv, safe])
        val = jnp.where(idx < n, val, -jnp.inf)
        s2 = plsc.sort_key_val(val, idx)
        p01 = _merge_top16(*s0, *s1)
        q = _merge_top16(*p01, *s2)
        top_v, top_i = _merge_top16(top_v, top_i, *q)

        mx = jnp.max(top_v)
        e = jnp.exp(top_v - mx)
        ssum = jnp.sum(e)
        soft = e / ssum

        pltpu.async_copy(h_hbm.at[top_i], hgat_v.at[slot], hsem)
        soft_v[slot, :] = soft

        @pl.when(t >= 2)
        def _():
            pltpu.make_async_copy(out_v.at[slot], att_hbm.at[r], osem).wait()
        prev_i = prev_v[slot, :]
        plsc.store_scatter(out_v, [slotv, prev_i], zeros16f)
        plsc.store_scatter(out_v, [slotv, top_i], soft)
        prev_v[slot, :] = top_i
        pltpu.async_copy(out_v.at[slot], att_hbm.at[r], osem)

        @pl.when(t >= 1)
        def _():
            pltpu.make_async_copy(h_hbm.at[top_i], hgat_v.at[nslot],
                                  hsem).wait()
            softp = soft_v[nslot, :]
            @pl.when(t >= 3)
            def _():
                pltpu.make_async_copy(hp_v.at[nslot], hp_hbm.at[r],
                                      hpsem).wait()
            accs = [jnp.zeros((L,), jnp.float32) for _ in range(8)]
            for k2 in range(K):
                w = jnp.take_along_axis(softp, jnp.full((L,), k2, jnp.int32),
                                        axis=0)
                for j in range(8):
                    accs[j] = accs[j] + w * hgat_v[nslot, k2, pl.ds(j * L, L)]
            for j in range(8):
                hp_v[nslot, pl.ds(j * L, L)] = accs[j]
            pltpu.async_copy(hp_v.at[nslot], hp_hbm.at[r - 1], hpsem)
        return 0

    lax.fori_loop(0, cnt, row_body, 0)

    lslot = lax.rem(cnt - 1, 2)
    rlast = base + cnt - 1
    pltpu.make_async_copy(h_hbm.at[jnp.zeros((L,), jnp.int32)],
                          hgat_v.at[lslot], hsem).wait()
    pltpu.make_async_copy(hp_v.at[lslot], hp_hbm.at[rlast], hpsem).wait()
    softp = soft_v[lslot, :]
    accs = [jnp.zeros((L,), jnp.float32) for _ in range(8)]
    for k2 in range(K):
        w = jnp.take_along_axis(softp, jnp.full((L,), k2, jnp.int32), axis=0)
        for j in range(8):
            accs[j] = accs[j] + w * hgat_v[lslot, k2, pl.ds(j * L, L)]
    for j in range(8):
        hp_v[lslot, pl.ds(j * L, L)] = accs[j]
    pltpu.async_copy(hp_v.at[lslot], hp_hbm.at[rlast], hpsem)

    pltpu.make_async_copy(out_v.at[0], att_hbm.at[base], osem).wait()
    pltpu.make_async_copy(out_v.at[0], att_hbm.at[base], osem).wait()
    pltpu.make_async_copy(hp_v.at[0], hp_hbm.at[base], hpsem).wait()
    pltpu.make_async_copy(hp_v.at[0], hp_hbm.at[base], hpsem).wait()


@functools.partial(jax.jit, static_argnames=())
def kernel(input, attention_logits, W):
    n_in, d_in = input.shape
    rows, n = attention_logits.shape
    d_out = W.shape[0]

    h = pl.pallas_call(
        _linear_kernel,
        out_shape=jax.ShapeDtypeStruct((n_in, d_out), jnp.float32),
    )(input, W)

    br = 200
    m = pl.pallas_call(
        _chunkmax_kernel,
        grid=(rows // br,),
        in_specs=[pl.BlockSpec((br, n), lambda i: (i, 0))],
        out_specs=pl.BlockSpec((br, CPAD), lambda i: (i, 0)),
        out_shape=jax.ShapeDtypeStruct((rows, CPAD), jnp.float32),
    )(attention_logits)

    mesh = plsc.VectorSubcoreMesh(core_axis_name="c", subcore_axis_name="s")
    att, hp = pl.kernel(
        _sc_body,
        out_type=[
            jax.ShapeDtypeStruct((rows, n), jnp.float32),
            jax.ShapeDtypeStruct((rows, d_out), jnp.float32),
        ],
        mesh=mesh,
        scratch_types=[
            pltpu.VMEM((2, n), jnp.float32),
            pltpu.VMEM((2, CPAD), jnp.float32),
            pltpu.VMEM((2, n), jnp.float32),
            pltpu.VMEM((2, K, d_out), jnp.float32),
            pltpu.VMEM((2, d_out), jnp.float32),
            pltpu.VMEM((2, L), jnp.int32),
            pltpu.VMEM((2, L), jnp.float32),
            pltpu.SemaphoreType.DMA,
            pltpu.SemaphoreType.DMA,
            pltpu.SemaphoreType.DMA,
            pltpu.SemaphoreType.DMA,
            pltpu.SemaphoreType.DMA,
        ],
        compiler_params=pltpu.CompilerParams(needs_layout_passes=False),
    )(attention_logits, m, h)

    oc = rows // n
    out = hp.reshape(oc, n, d_out).transpose(1, 0, 2).reshape(n, oc * d_out)
    return out, att


# speedup vs baseline: 1.6155x; 1.0017x over previous
import functools

import jax
import jax.numpy as jnp
from jax import lax
from jax.experimental import pallas as pl
from jax.experimental.pallas import tpu as pltpu
from jax.experimental.pallas import tpu_sc as plsc

K = 16
L = 16
NC = 2
NS = 16
NW = NC * NS
C = 128
DEPTH = 78
CPAD = 128


def _linear_kernel(x_ref, w_ref, h_ref):
    h_ref[...] = lax.dot_general(
        x_ref[...], w_ref[...],
        (((1,), (1,)), ((), ())),
        preferred_element_type=jnp.float32,
    )


def _chunkmax_kernel(x_ref, m_ref):
    x = x_ref[...]
    br, n = x.shape
    m = x[:, 0:C]
    for v in range(1, n // C):
        m = jnp.maximum(m, x[:, C * v:C * (v + 1)])
    tail = jnp.concatenate(
        [x[:, (n // C) * C:],
         jnp.full((br, C - n % C), -jnp.inf, jnp.float32)], axis=1)
    m_ref[...] = jnp.maximum(m, tail)


def _merge_top16(tv, ti, sk, si):
    rb = lax.rev(sk, (0,))
    rbi = lax.rev(si, (0,))
    keep = tv >= rb
    mv = jnp.maximum(tv, rb)
    mi = jnp.where(keep, ti, rbi)
    return plsc.sort_key_val(mv, mi)


def _iota16():
    return lax.iota(jnp.int32, L)


def _sc_body(logits_hbm, m_hbm, h_hbm, att_hbm, hp_hbm,
             row_v, m_v, out_v, hgat_v, hp_v, prev_v, soft_v,
             rsem, msem, osem, hpsem, hsem):
    n = 10000
    wid = lax.axis_index("s") * NC + lax.axis_index("c")
    n_lo = n // NW
    n_extra = n - n_lo * NW
    cnt = jnp.where(wid < n_extra, n_lo + 1, n_lo)
    base = jnp.where(wid < n_extra, wid * (n_lo + 1),
                     n_extra * (n_lo + 1) + (wid - n_extra) * n_lo)
    last = n - 1

    def zero_body(i, _):
        out_v[0, pl.ds(i * L, L)] = jnp.zeros((L,), jnp.float32)
        out_v[1, pl.ds(i * L, L)] = jnp.zeros((L,), jnp.float32)
        return 0
    lax.fori_loop(0, n // L, zero_body, 0)

    prev_v[0, :] = jnp.zeros((L,), jnp.int32)
    prev_v[1, :] = jnp.zeros((L,), jnp.int32)

    zeros16f = jnp.zeros((L,), jnp.float32)
    iota = _iota16()

    pltpu.async_copy(logits_hbm.at[base], row_v.at[0], rsem)
    pltpu.async_copy(m_hbm.at[base], m_v.at[0], msem)

    def row_body(t, _):
        r = base + t
        slot = lax.rem(t, 2)
        nslot = 1 - slot
        rnext = jnp.minimum(base + t + 1, last)

        pltpu.make_async_copy(logits_hbm.at[r], row_v.at[slot], rsem).wait()
        pltpu.make_async_copy(m_hbm.at[r], m_v.at[slot], msem).wait()

        @pl.when(t + 1 < cnt)
        def _():
            pltpu.async_copy(logits_hbm.at[rnext], row_v.at[nslot], rsem)
            pltpu.async_copy(m_hbm.at[rnext], m_v.at[nslot], msem)

        slotv = jnp.full((L,), slot, jnp.int32)

        leaves = []
        for v in range(CPAD // L):
            k = m_v[slot, pl.ds(v * L, L)]
            leaves.append(plsc.sort_key_val(k, iota + v * L))
        while len(leaves) > 1:
            nxt = [_merge_top16(*leaves[i], *leaves[i + 1])
                   for i in range(0, len(leaves), 2)]
            leaves = nxt
        tv, ti = leaves[0]

        top_v = jnp.full((L,), -jnp.inf, jnp.float32)
        top_i = jnp.zeros((L,), jnp.int32)

        def leaf(j):
            idx = ti + C * j
            val = plsc.load_gather(row_v, [slotv, idx])
            return plsc.sort_key_val(val, idx)

        def chunk_body(t2, carry):
            cv, ci = carry
            j0 = 8 * t2
            ss = [leaf(j0 + k) for k in range(8)]
            while len(ss) > 1:
                ss = [_merge_top16(*ss[i], *ss[i + 1])
                      for i in range(0, len(ss), 2)]
            cv, ci = _merge_top16(cv, ci, *ss[0])
            return cv, ci

        top_v, top_i = lax.fori_loop(0, DEPTH // 8, chunk_body,
                                     (top_v, top_i))
        ss = [leaf(j) for j in range(8 * (DEPTH // 8), DEPTH)]
        idx = ti + C * DEPTH
        safe = jnp.minimum(idx, n - 1)
        val = plsc.load_gather(row_v, [slotv, safe])
        val = jnp.where(idx < n, val, -jnp.inf)
        ss.append(plsc.sort_key_val(val, idx))
        while len(ss) > 1:
            rest = ss[2:]
            rest.append(_merge_top16(*ss[0], *ss[1]))
            ss = rest
        top_v, top_i = _merge_top16(top_v, top_i, *ss[0])

        mx = jnp.max(top_v)
        e = jnp.exp(top_v - mx)
        ssum = jnp.sum(e)
        soft = e / ssum

        pltpu.async_copy(h_hbm.at[top_i], hgat_v.at[slot], hsem)
        soft_v[slot, :] = soft

        @pl.when(t >= 2)
        def _():
            pltpu.make_async_copy(out_v.at[slot], att_hbm.at[r], osem).wait()
        prev_i = prev_v[slot, :]
        plsc.store_scatter(out_v, [slotv, prev_i], zeros16f)
        plsc.store_scatter(out_v, [slotv, top_i], soft)
        prev_v[slot, :] = top_i
        pltpu.async_copy(out_v.at[slot], att_hbm.at[r], osem)

        @pl.when(t >= 1)
        def _():
            pltpu.make_async_copy(h_hbm.at[top_i], hgat_v.at[nslot],
                                  hsem).wait()
            softp = soft_v[nslot, :]
            @pl.when(t >= 3)
            def _():
                pltpu.make_async_copy(hp_v.at[nslot], hp_hbm.at[r],
                                      hpsem).wait()
            accs = [jnp.zeros((L,), jnp.float32) for _ in range(8)]
            for k2 in range(K):
                w = jnp.take_along_axis(softp, jnp.full((L,), k2, jnp.int32),
                                        axis=0)
                for j in range(8):
                    accs[j] = accs[j] + w * hgat_v[nslot, k2, pl.ds(j * L, L)]
            for j in range(8):
                hp_v[nslot, pl.ds(j * L, L)] = accs[j]
            pltpu.async_copy(hp_v.at[nslot], hp_hbm.at[r - 1], hpsem)
        return 0

    lax.fori_loop(0, cnt, row_body, 0)

    lslot = lax.rem(cnt - 1, 2)
    rlast = base + cnt - 1
    pltpu.make_async_copy(h_hbm.at[jnp.zeros((L,), jnp.int32)],
                          hgat_v.at[lslot], hsem).wait()
    pltpu.make_async_copy(hp_v.at[lslot], hp_hbm.at[rlast], hpsem).wait()
    softp = soft_v[lslot, :]
    accs = [jnp.zeros((L,), jnp.float32) for _ in range(8)]
    for k2 in range(K):
        w = jnp.take_along_axis(softp, jnp.full((L,), k2, jnp.int32), axis=0)
        for j in range(8):
            accs[j] = accs[j] + w * hgat_v[lslot, k2, pl.ds(j * L, L)]
    for j in range(8):
        hp_v[lslot, pl.ds(j * L, L)] = accs[j]
    pltpu.async_copy(hp_v.at[lslot], hp_hbm.at[rlast], hpsem)

    pltpu.make_async_copy(out_v.at[0], att_hbm.at[base], osem).wait()
    pltpu.make_async_copy(out_v.at[0], att_hbm.at[base], osem).wait()
    pltpu.make_async_copy(hp_v.at[0], hp_hbm.at[base], hpsem).wait()
    pltpu.make_async_copy(hp_v.at[0], hp_hbm.at[base], hpsem).wait()


@functools.partial(jax.jit, static_argnames=())
def kernel(input, attention_logits, W):
    n_in, d_in = input.shape
    rows, n = attention_logits.shape
    d_out = W.shape[0]

    h = pl.pallas_call(
        _linear_kernel,
        out_shape=jax.ShapeDtypeStruct((n_in, d_out), jnp.float32),
    )(input, W)

    br = 200
    m = pl.pallas_call(
        _chunkmax_kernel,
        grid=(rows // br,),
        in_specs=[pl.BlockSpec((br, n), lambda i: (i, 0))],
        out_specs=pl.BlockSpec((br, CPAD), lambda i: (i, 0)),
        out_shape=jax.ShapeDtypeStruct((rows, CPAD), jnp.float32),
    )(attention_logits)

    mesh = plsc.VectorSubcoreMesh(core_axis_name="c", subcore_axis_name="s")
    att, hp = pl.kernel(
        _sc_body,
        out_type=[
            jax.ShapeDtypeStruct((rows, n), jnp.float32),
            jax.ShapeDtypeStruct((rows, d_out), jnp.float32),
        ],
        mesh=mesh,
        scratch_types=[
            pltpu.VMEM((2, n), jnp.float32),
            pltpu.VMEM((2, CPAD), jnp.float32),
            pltpu.VMEM((2, n), jnp.float32),
            pltpu.VMEM((2, K, d_out), jnp.float32),
            pltpu.VMEM((2, d_out), jnp.float32),
            pltpu.VMEM((2, L), jnp.int32),
            pltpu.VMEM((2, L), jnp.float32),
            pltpu.SemaphoreType.DMA,
            pltpu.SemaphoreType.DMA,
            pltpu.SemaphoreType.DMA,
            pltpu.SemaphoreType.DMA,
            pltpu.SemaphoreType.DMA,
        ],
        compiler_params=pltpu.CompilerParams(needs_layout_passes=False),
    )(attention_logits, m, h)

    oc = rows // n
    out = hp.reshape(oc, n, d_out).transpose(1, 0, 2).reshape(n, oc * d_out)
    return out, att


# speedup vs baseline: 1.6250x; 1.0059x over previous
import functools

import jax
import jax.numpy as jnp
from jax import lax
from jax.experimental import pallas as pl
from jax.experimental.pallas import tpu as pltpu
from jax.experimental.pallas import tpu_sc as plsc

K = 16
L = 16
NC = 2
NS = 16
NW = NC * NS
C = 128
DEPTH = 78
CPAD = 128


def _tc_kernel(in_ref, x_ref, w_ref, h_ref, m_ref):
    h_ref[...] = lax.dot_general(
        in_ref[...], w_ref[...],
        (((1,), (1,)), ((), ())),
        preferred_element_type=jnp.float32,
    )
    x = x_ref[...]
    br, n = x.shape
    m = x[:, 0:C]
    for v in range(1, n // C):
        m = jnp.maximum(m, x[:, C * v:C * (v + 1)])
    tail = jnp.concatenate(
        [x[:, (n // C) * C:],
         jnp.full((br, C - n % C), -jnp.inf, jnp.float32)], axis=1)
    m_ref[...] = jnp.maximum(m, tail)


def _merge_top16(tv, ti, sk, si):
    rb = lax.rev(sk, (0,))
    rbi = lax.rev(si, (0,))
    keep = tv >= rb
    mv = jnp.maximum(tv, rb)
    mi = jnp.where(keep, ti, rbi)
    return plsc.sort_key_val(mv, mi)


def _iota16():
    return lax.iota(jnp.int32, L)


def _sc_body(logits_hbm, m_hbm, h_hbm, att_hbm, hp_hbm,
             row_v, m_v, out_v, hgat_v, hp_v, prev_v, soft_v,
             rsem, msem, osem, hpsem, hsem):
    n = 10000
    wid = lax.axis_index("s") * NC + lax.axis_index("c")
    n_lo = n // NW
    n_extra = n - n_lo * NW
    cnt = jnp.where(wid < n_extra, n_lo + 1, n_lo)
    base = jnp.where(wid < n_extra, wid * (n_lo + 1),
                     n_extra * (n_lo + 1) + (wid - n_extra) * n_lo)
    last = n - 1

    def zero_body(i, _):
        out_v[0, pl.ds(i * L, L)] = jnp.zeros((L,), jnp.float32)
        out_v[1, pl.ds(i * L, L)] = jnp.zeros((L,), jnp.float32)
        return 0
    lax.fori_loop(0, n // L, zero_body, 0)

    prev_v[0, :] = jnp.zeros((L,), jnp.int32)
    prev_v[1, :] = jnp.zeros((L,), jnp.int32)

    zeros16f = jnp.zeros((L,), jnp.float32)
    iota = _iota16()

    pltpu.async_copy(logits_hbm.at[base], row_v.at[0], rsem)
    pltpu.async_copy(m_hbm.at[base], m_v.at[0], msem)

    def row_body(t, _):
        r = base + t
        slot = lax.rem(t, 2)
        nslot = 1 - slot
        rnext = jnp.minimum(base + t + 1, last)

        pltpu.make_async_copy(logits_hbm.at[r], row_v.at[slot], rsem).wait()
        pltpu.make_async_copy(m_hbm.at[r], m_v.at[slot], msem).wait()

        @pl.when(t + 1 < cnt)
        def _():
            pltpu.async_copy(logits_hbm.at[rnext], row_v.at[nslot], rsem)
            pltpu.async_copy(m_hbm.at[rnext], m_v.at[nslot], msem)

        slotv = jnp.full((L,), slot, jnp.int32)

        leaves = []
        for v in range(CPAD // L):
            k = m_v[slot, pl.ds(v * L, L)]
            leaves.append(plsc.sort_key_val(k, iota + v * L))
        while len(leaves) > 1:
            nxt = [_merge_top16(*leaves[i], *leaves[i + 1])
                   for i in range(0, len(leaves), 2)]
            leaves = nxt
        tv, ti = leaves[0]

        top_v = jnp.full((L,), -jnp.inf, jnp.float32)
        top_i = jnp.zeros((L,), jnp.int32)

        def leaf(j):
            idx = ti + C * j
            val = plsc.load_gather(row_v, [slotv, idx])
            return plsc.sort_key_val(val, idx)

        def chunk_body(t2, carry):
            cv, ci = carry
            j0 = 8 * t2
            ss = [leaf(j0 + k) for k in range(8)]
            while len(ss) > 1:
                ss = [_merge_top16(*ss[i], *ss[i + 1])
                      for i in range(0, len(ss), 2)]
            cv, ci = _merge_top16(cv, ci, *ss[0])
            return cv, ci

        top_v, top_i = lax.fori_loop(0, DEPTH // 8, chunk_body,
                                     (top_v, top_i))
        ss = [leaf(j) for j in range(8 * (DEPTH // 8), DEPTH)]
        idx = ti + C * DEPTH
        safe = jnp.minimum(idx, n - 1)
        val = plsc.load_gather(row_v, [slotv, safe])
        val = jnp.where(idx < n, val, -jnp.inf)
        ss.append(plsc.sort_key_val(val, idx))
        while len(ss) > 1:
            rest = ss[2:]
            rest.append(_merge_top16(*ss[0], *ss[1]))
            ss = rest
        top_v, top_i = _merge_top16(top_v, top_i, *ss[0])

        mx = jnp.max(top_v)
        e = jnp.exp(top_v - mx)
        ssum = jnp.sum(e)
        soft = e / ssum

        pltpu.async_copy(h_hbm.at[top_i], hgat_v.at[slot], hsem)
        soft_v[slot, :] = soft

        @pl.when(t >= 2)
        def _():
            pltpu.make_async_copy(out_v.at[slot], att_hbm.at[r], osem).wait()
        prev_i = prev_v[slot, :]
        plsc.store_scatter(out_v, [slotv, prev_i], zeros16f)
        plsc.store_scatter(out_v, [slotv, top_i], soft)
        prev_v[slot, :] = top_i
        pltpu.async_copy(out_v.at[slot], att_hbm.at[r], osem)

        @pl.when(t >= 1)
        def _():
            pltpu.make_async_copy(h_hbm.at[top_i], hgat_v.at[nslot],
                                  hsem).wait()
            softp = soft_v[nslot, :]
            @pl.when(t >= 3)
            def _():
                pltpu.make_async_copy(hp_v.at[nslot], hp_hbm.at[r],
                                      hpsem).wait()
            accs = [jnp.zeros((L,), jnp.float32) for _ in range(8)]
            for k2 in range(K):
                w = jnp.take_along_axis(softp, jnp.full((L,), k2, jnp.int32),
                                        axis=0)
                for j in range(8):
                    accs[j] = accs[j] + w * hgat_v[nslot, k2, pl.ds(j * L, L)]
            for j in range(8):
                hp_v[nslot, pl.ds(j * L, L)] = accs[j]
            pltpu.async_copy(hp_v.at[nslot], hp_hbm.at[r - 1], hpsem)
        return 0

    lax.fori_loop(0, cnt, row_body, 0)

    lslot = lax.rem(cnt - 1, 2)
    rlast = base + cnt - 1
    pltpu.make_async_copy(h_hbm.at[jnp.zeros((L,), jnp.int32)],
                          hgat_v.at[lslot], hsem).wait()
    pltpu.make_async_copy(hp_v.at[lslot], hp_hbm.at[rlast], hpsem).wait()
    softp = soft_v[lslot, :]
    accs = [jnp.zeros((L,), jnp.float32) for _ in range(8)]
    for k2 in range(K):
        w = jnp.take_along_axis(softp, jnp.full((L,), k2, jnp.int32), axis=0)
        for j in range(8):
            accs[j] = accs[j] + w * hgat_v[lslot, k2, pl.ds(j * L, L)]
    for j in range(8):
        hp_v[lslot, pl.ds(j * L, L)] = accs[j]
    pltpu.async_copy(hp_v.at[lslot], hp_hbm.at[rlast], hpsem)

    pltpu.make_async_copy(out_v.at[0], att_hbm.at[base], osem).wait()
    pltpu.make_async_copy(out_v.at[0], att_hbm.at[base], osem).wait()
    pltpu.make_async_copy(hp_v.at[0], hp_hbm.at[base], hpsem).wait()
    pltpu.make_async_copy(hp_v.at[0], hp_hbm.at[base], hpsem).wait()


@functools.partial(jax.jit, static_argnames=())
def kernel(input, attention_logits, W):
    n_in, d_in = input.shape
    rows, n = attention_logits.shape
    d_out = W.shape[0]

    br = 200
    h, m = pl.pallas_call(
        _tc_kernel,
        grid=(rows // br,),
        in_specs=[
            pl.BlockSpec((br, d_in), lambda i: (i, 0)),
            pl.BlockSpec((br, n), lambda i: (i, 0)),
            pl.BlockSpec((d_out, d_in), lambda i: (0, 0)),
        ],
        out_specs=[
            pl.BlockSpec((br, d_out), lambda i: (i, 0)),
            pl.BlockSpec((br, CPAD), lambda i: (i, 0)),
        ],
        out_shape=[
            jax.ShapeDtypeStruct((n_in, d_out), jnp.float32),
            jax.ShapeDtypeStruct((rows, CPAD), jnp.float32),
        ],
    )(input, attention_logits, W)

    mesh = plsc.VectorSubcoreMesh(core_axis_name="c", subcore_axis_name="s")
    att, hp = pl.kernel(
        _sc_body,
        out_type=[
            jax.ShapeDtypeStruct((rows, n), jnp.float32),
            jax.ShapeDtypeStruct((rows, d_out), jnp.float32),
        ],
        mesh=mesh,
        scratch_types=[
            pltpu.VMEM((2, n), jnp.float32),
            pltpu.VMEM((2, CPAD), jnp.float32),
            pltpu.VMEM((2, n), jnp.float32),
            pltpu.VMEM((2, K, d_out), jnp.float32),
            pltpu.VMEM((2, d_out), jnp.float32),
            pltpu.VMEM((2, L), jnp.int32),
            pltpu.VMEM((2, L), jnp.float32),
            pltpu.SemaphoreType.DMA,
            pltpu.SemaphoreType.DMA,
            pltpu.SemaphoreType.DMA,
            pltpu.SemaphoreType.DMA,
            pltpu.SemaphoreType.DMA,
        ],
        compiler_params=pltpu.CompilerParams(needs_layout_passes=False),
    )(attention_logits, m, h)

    oc = rows // n
    out = hp.reshape(oc, n, d_out).transpose(1, 0, 2).reshape(n, oc * d_out)
    return out, att
